# baseline (speedup 1.0000x reference)
import functools

import jax
import jax.numpy as jnp
from jax import lax
from jax.experimental import pallas as pl
from jax.experimental.pallas import tpu as pltpu
from jax.experimental.pallas import tpu_sc as plsc

_NC = 2
_NS = 16
_L = 16
_NBUF = 8
_CHUNK = 32
_RS = 6144


def _fm_body(F, FP, K, BPW, BPW2, B2OFF, cat_hbm, emb_hbm, lin_hbm, bias_hbm,
             out_hbm, idx_v, out_v, bias_v, *bufs):
  ebufs = bufs[:_NBUF]
  lbufs = bufs[_NBUF:2 * _NBUF]
  esems = bufs[2 * _NBUF:3 * _NBUF]
  lsems = bufs[3 * _NBUF:4 * _NBUF]
  KG = K // _L

  wid = lax.axis_index("s") * _NC + lax.axis_index("c")
  base = wid * BPW

  pltpu.sync_copy(cat_hbm.at[pl.ds(base, _CHUNK)], idx_v.at[pl.ds(0, _CHUNK)])
  pltpu.sync_copy(bias_hbm, bias_v)
  bias0 = bias_v[...][0]

  def _islot(j):
    return lax.bitwise_and(lax.shift_right_logical(j, 5), 1) * _CHUNK + \
        lax.bitwise_and(j, _CHUNK - 1)

  def _fire(j, b):
    row = idx_v.at[_islot(j)]
    pltpu.async_copy(emb_hbm.at[row], ebufs[b], esems[b])
    pltpu.async_copy(lin_hbm.at[row], lbufs[b], lsems[b])

  for b in range(_NBUF):
    _fire(jnp.int32(b), b)

  zero = jnp.zeros((_L,), jnp.float32)
  lane_iota = lax.iota(jnp.int32, _L)

  def _gstep(g, resvec):
    c = lax.div(g, _CHUNK // _NBUF)

    @pl.when((lax.rem(g, _CHUNK // _NBUF) == 0) & (c < BPW // _CHUNK - 1))
    def _():
      dst = lax.bitwise_and(c + 1, 1) * _CHUNK
      pltpu.sync_copy(cat_hbm.at[pl.ds(base + (c + 1) * _CHUNK, _CHUNK)],
                      idx_v.at[pl.ds(dst, _CHUNK)])

    for b in range(_NBUF):
      j = g * _NBUF + b
      row = idx_v.at[_islot(j)]
      pltpu.make_async_copy(emb_hbm.at[row], ebufs[b], esems[b]).wait()
      pltpu.make_async_copy(lin_hbm.at[row], lbufs[b], lsems[b]).wait()

      def _accum(f, cr):
        ss = cr[:KG]
        qq = cr[KG:2 * KG]
        ls = cr[2 * KG]
        new_ss = []
        new_qq = []
        for gg in range(KG):
          e = ebufs[b][f, pl.ds(gg * _L, _L)]
          new_ss.append(ss[gg] + e)
          new_qq.append(qq[gg] + e * e)
        return (*new_ss, *new_qq, ls + lbufs[b][f])

      res = lax.fori_loop(0, F, _accum, (zero,) * (2 * KG + 1))
      ss = res[:KG]
      qq = res[KG:2 * KG]
      ls = res[2 * KG]
      r = zero
      for gg in range(KG):
        r = r + (ss[gg] * ss[gg] - qq[gg])
      v = 0.5 * r + ls
      total = bias0
      for lane_i in range(_L):
        total = total + v[lane_i]
      lane = lax.rem(j, _L)
      resvec = jnp.where(lane_iota == lane, total, resvec)
      if b == _NBUF - 1:

        @pl.when(lane == _L - 1)
        def _():
          out_v[pl.ds(j - (_L - 1), _L)] = resvec

      nj = j + _NBUF

      @pl.when(nj < BPW)
      def _():
        _fire(nj, b)

    return resvec

  lax.fori_loop(0, BPW // _NBUF, _gstep, zero)
  pltpu.sync_copy(out_v.at[pl.ds(0, BPW)], out_hbm.at[pl.ds(base, BPW)])

  base2 = B2OFF + wid * BPW2

  pltpu.sync_copy(cat_hbm.at[pl.ds(base2, _CHUNK)], idx_v.at[pl.ds(0, _CHUNK)])

  def _fire2(j, b):
    pltpu.async_copy(lin_hbm.at[idx_v.at[_islot(j)]], lbufs[b], lsems[b])

  for b in range(_NBUF):
    _fire2(jnp.int32(b), b)

  def _gstep2(g, resvec):
    c = lax.div(g, _CHUNK // _NBUF)

    @pl.when((lax.rem(g, _CHUNK // _NBUF) == 0) & (c < BPW2 // _CHUNK - 1))
    def _():
      dst = lax.bitwise_and(c + 1, 1) * _CHUNK
      pltpu.sync_copy(cat_hbm.at[pl.ds(base2 + (c + 1) * _CHUNK, _CHUNK)],
                      idx_v.at[pl.ds(dst, _CHUNK)])

    for b in range(_NBUF):
      j = g * _NBUF + b
      pltpu.make_async_copy(
          lin_hbm.at[idx_v.at[_islot(j)]], lbufs[b], lsems[b]).wait()

      def _accum(f, ls):
        return ls + lbufs[b][f]

      ls = lax.fori_loop(0, F, _accum, zero)
      total = bias0 + ls[0]
      lane = lax.rem(j, _L)
      resvec = jnp.where(lane_iota == lane, total, resvec)
      if b == _NBUF - 1:

        @pl.when(lane == _L - 1)
        def _():
          out_v[pl.ds(j - (_L - 1), _L)] = resvec

      nj = j + _NBUF

      @pl.when(nj < BPW2)
      def _():
        _fire2(nj, b)

    return resvec

  lax.fori_loop(0, BPW2 // _NBUF, _gstep2, zero)
  pltpu.sync_copy(out_v.at[pl.ds(0, BPW2)], out_hbm.at[pl.ds(base2, BPW2)])


def _tc_fm(emb_pad, cat_tc, F, K):
  NR, FP = cat_tc.shape
  RB = 8
  VP = emb_pad.shape[0]

  def body(idx_ref, tab_ref, out_ref):
    outs = []
    for r in range(RB):
      def _accum(f, cr):
        s, q = cr
        e = tab_ref[pl.ds(idx_ref[r, f], 1), :]
        return (s + e, q + e * e)

      zz = jnp.zeros((1, K), jnp.float32)
      s, q = lax.fori_loop(0, F, _accum, (zz, zz))
      fm = 0.5 * (jnp.sum(s * s) - jnp.sum(q))
      outs.append(jnp.full((1, 128), fm, jnp.float32))
    out_ref[...] = jnp.concatenate(outs, axis=0)

  return pl.pallas_call(
      body,
      grid=(NR // RB,),
      in_specs=[
          pl.BlockSpec((RB, FP), lambda i: (i, 0),
                       memory_space=pltpu.SMEM),
          pl.BlockSpec((VP, K), lambda i: (0, 0)),
      ],
      out_specs=pl.BlockSpec((RB, 128), lambda i: (i, 0)),
      out_shape=jax.ShapeDtypeStruct((NR, 128), jnp.float32),
  )(cat_tc, emb_pad)


def kernel(cat_features, emb_table, lin_table, bias):
  B, F = cat_features.shape
  V, K = emb_table.shape
  NW = _NC * _NS
  BPW = _RS // NW
  BPW2 = (B - _RS) // NW
  FP = -(-F // 8) * 8

  cat_pad = jnp.pad(cat_features, ((0, 0), (0, FP - F)))
  lin_pad = jnp.pad(lin_table, ((0, 0), (0, _L - lin_table.shape[1])))
  bias_pad = jnp.pad(bias, (0, _L - bias.shape[0]))
  emb_pad = jnp.pad(emb_table, ((0, -(-V // 8) * 8 - V), (0, 0)))

  mesh = plsc.VectorSubcoreMesh(core_axis_name="c", subcore_axis_name="s")
  scratch = [
      pltpu.VMEM((2 * _CHUNK, FP), jnp.int32),
      pltpu.VMEM((max(BPW, BPW2),), jnp.float32),
      pltpu.VMEM((_L,), jnp.float32),
  ]
  scratch += [pltpu.VMEM((FP, K), jnp.float32) for _ in range(_NBUF)]
  scratch += [pltpu.VMEM((FP, _L), jnp.float32) for _ in range(_NBUF)]
  scratch += [pltpu.SemaphoreType.DMA for _ in range(2 * _NBUF)]

  body = functools.partial(_fm_body, F, FP, K, BPW, BPW2, _RS)
  out_sc = pl.kernel(
      body,
      out_type=jax.ShapeDtypeStruct((B,), jnp.float32),
      mesh=mesh,
      scratch_types=scratch,
      compiler_params=pltpu.CompilerParams(use_tc_tiling_on_sc=False),
  )(cat_pad, emb_table, lin_pad, bias_pad)

  fm_tc = _tc_fm(emb_pad, cat_pad[_RS:], F, K)[:, 0]
  out = jnp.concatenate([out_sc[:_RS], out_sc[_RS:] + fm_tc])
  return out.reshape(B, 1)


# speedup vs baseline: 3.4908x; 3.4908x over previous
import functools

import jax
import jax.numpy as jnp
from jax import lax
from jax.experimental import pallas as pl
from jax.experimental.pallas import tpu as pltpu
from jax.experimental.pallas import tpu_sc as plsc

_NC = 2
_NS = 16
_L = 16
_NBUF = 8
_CHUNK = 32


def _fm_body(F, FP, K, BPW, cat_hbm, emb_hbm, lin_hbm, bias_hbm, out_hbm,
             idx_v, out_v, bias_v, *bufs):
  ebufs = bufs[:_NBUF]
  lbufs = bufs[_NBUF:2 * _NBUF]
  esems = bufs[2 * _NBUF:3 * _NBUF]
  lsems = bufs[3 * _NBUF:4 * _NBUF]
  KG = K // _L

  wid = lax.axis_index("s") * _NC + lax.axis_index("c")
  base = wid * BPW

  pltpu.sync_copy(cat_hbm.at[pl.ds(base, _CHUNK)], idx_v.at[pl.ds(0, _CHUNK)])
  pltpu.sync_copy(bias_hbm, bias_v)
  bias0 = bias_v[...][0]

  def _islot(j):
    return lax.bitwise_and(lax.shift_right_logical(j, 5), 1) * _CHUNK + \
        lax.bitwise_and(j, _CHUNK - 1)

  def _fire(j, b):
    row = idx_v.at[_islot(j)]
    pltpu.async_copy(emb_hbm.at[row], ebufs[b], esems[b])
    pltpu.async_copy(lin_hbm.at[row], lbufs[b], lsems[b])

  for b in range(_NBUF):
    _fire(jnp.int32(b), b)

  zero = jnp.zeros((_L,), jnp.float32)
  lane_iota = lax.iota(jnp.int32, _L)

  def _gstep(g, resvec):
    c = lax.div(g, _CHUNK // _NBUF)

    @pl.when((lax.rem(g, _CHUNK // _NBUF) == 0) & (c < BPW // _CHUNK - 1))
    def _():
      dst = lax.bitwise_and(c + 1, 1) * _CHUNK
      pltpu.sync_copy(cat_hbm.at[pl.ds(base + (c + 1) * _CHUNK, _CHUNK)],
                      idx_v.at[pl.ds(dst, _CHUNK)])

    for b in range(_NBUF):
      j = g * _NBUF + b
      row = idx_v.at[_islot(j)]
      pltpu.make_async_copy(emb_hbm.at[row], ebufs[b], esems[b]).wait()
      pltpu.make_async_copy(lin_hbm.at[row], lbufs[b], lsems[b]).wait()

      def _accum(f, carry):
        ss = carry[:KG]
        qq = carry[KG:2 * KG]
        ls = carry[2 * KG]
        new_ss = []
        new_qq = []
        for gg in range(KG):
          e = ebufs[b][f, pl.ds(gg * _L, _L)]
          new_ss.append(ss[gg] + e)
          new_qq.append(qq[gg] + e * e)
        return (*new_ss, *new_qq, ls + lbufs[b][f])

      res = lax.fori_loop(0, F, _accum, (zero,) * (2 * KG + 1))
      ss = res[:KG]
      qq = res[KG:2 * KG]
      ls = res[2 * KG]
      r = zero
      for gg in range(KG):
        r = r + (ss[gg] * ss[gg] - qq[gg])
      v = 0.5 * r + ls
      total = bias0
      for lane_i in range(_L):
        total = total + v[lane_i]
      lane = lax.rem(j, _L)
      resvec = jnp.where(lane_iota == lane, total, resvec)
      if (b + 1) % _L == 0 or _NBUF < _L and b == _NBUF - 1:

        @pl.when(lane == _L - 1)
        def _():
          out_v[pl.ds(j - (_L - 1), _L)] = resvec

      nj = j + _NBUF

      @pl.when(nj < BPW)
      def _():
        _fire(nj, b)

    return resvec

  lax.fori_loop(0, BPW // _NBUF, _gstep, zero)
  pltpu.sync_copy(out_v, out_hbm.at[pl.ds(base, BPW)])


def kernel(cat_features, emb_table, lin_table, bias):
  B, F = cat_features.shape
  V, K = emb_table.shape
  NW = _NC * _NS
  BPW = B // NW
  FP = -(-F // 8) * 8

  cat_pad = jnp.pad(cat_features, ((0, 0), (0, FP - F)))
  lin_pad = jnp.pad(lin_table, ((0, 0), (0, _L - lin_table.shape[1])))
  bias_pad = jnp.pad(bias, (0, _L - bias.shape[0]))

  mesh = plsc.VectorSubcoreMesh(core_axis_name="c", subcore_axis_name="s")
  scratch = [
      pltpu.VMEM((2 * _CHUNK, FP), jnp.int32),
      pltpu.VMEM((BPW,), jnp.float32),
      pltpu.VMEM((_L,), jnp.float32),
  ]
  scratch += [pltpu.VMEM((FP, K), jnp.float32) for _ in range(_NBUF)]
  scratch += [pltpu.VMEM((FP, _L), jnp.float32) for _ in range(_NBUF)]
  scratch += [pltpu.SemaphoreType.DMA for _ in range(2 * _NBUF)]

  body = functools.partial(_fm_body, F, FP, K, BPW)
  out = pl.kernel(
      body,
      out_type=jax.ShapeDtypeStruct((B,), jnp.float32),
      mesh=mesh,
      scratch_types=scratch,
      compiler_params=pltpu.CompilerParams(use_tc_tiling_on_sc=False),
  )(cat_pad, emb_table, lin_pad, bias_pad)
  return out.reshape(B, 1)


# speedup vs baseline: 6.0582x; 1.7355x over previous
import functools

import jax
import jax.numpy as jnp
from jax import lax
from jax.experimental import pallas as pl
from jax.experimental.pallas import tpu as pltpu
from jax.experimental.pallas import tpu_sc as plsc

_NC = 2
_NS = 16
_L = 16
_NBUF = 8
_CHUNK = 32


def _fm_body(F, FP, K, BPW, cat_hbm, emb_hbm, lin_hbm, bias_hbm, out_hbm,
             idx_v, out_v, bias_v, *bufs):
  ebufs = bufs[:_NBUF]
  lbufs = bufs[_NBUF:2 * _NBUF]
  esems = bufs[2 * _NBUF:3 * _NBUF]
  lsems = bufs[3 * _NBUF:4 * _NBUF]
  KG = K // _L

  wid = lax.axis_index("s") * _NC + lax.axis_index("c")
  base = wid * BPW

  pltpu.sync_copy(cat_hbm.at[pl.ds(base, _CHUNK)], idx_v.at[pl.ds(0, _CHUNK)])
  pltpu.sync_copy(bias_hbm, bias_v)
  bias0 = bias_v[...][0]

  def _islot(j):
    return lax.bitwise_and(lax.shift_right_logical(j, 5), 1) * _CHUNK + \
        lax.bitwise_and(j, _CHUNK - 1)

  def _fire(j, b):
    row = idx_v.at[_islot(j)]
    pltpu.async_copy(emb_hbm.at[row], ebufs[b], esems[b])
    pltpu.async_copy(lin_hbm.at[row], lbufs[b], lsems[b])

  for b in range(_NBUF):
    _fire(jnp.int32(b), b)

  zero = jnp.zeros((_L,), jnp.float32)
  lane_iota = lax.iota(jnp.int32, _L)

  def _gstep(g, resvec):
    c = lax.div(g, _CHUNK // _NBUF)

    @pl.when((lax.rem(g, _CHUNK // _NBUF) == 0) & (c < BPW // _CHUNK - 1))
    def _():
      dst = lax.bitwise_and(c + 1, 1) * _CHUNK
      pltpu.sync_copy(cat_hbm.at[pl.ds(base + (c + 1) * _CHUNK, _CHUNK)],
                      idx_v.at[pl.ds(dst, _CHUNK)])

    for b in range(_NBUF):
      j = g * _NBUF + b
      row = idx_v.at[_islot(j)]
      pltpu.make_async_copy(emb_hbm.at[row], ebufs[b], esems[b]).wait()
      pltpu.make_async_copy(lin_hbm.at[row], lbufs[b], lsems[b]).wait()

      def _accum(f, carry):
        ss = carry[:KG]
        qq = carry[KG:2 * KG]
        ls = carry[2 * KG]
        new_ss = []
        new_qq = []
        for g2 in range(KG // 2):
          pk = ebufs[b][f, pl.ds(g2 * 2 * _L, 2 * _L)]
          ea, eb = plsc.unpack(pk, format=plsc.PackFormat.INTERLEAVED,
                               preferred_element_type=jnp.float32)
          for gg, e in ((2 * g2, ea), (2 * g2 + 1, eb)):
            new_ss.append(ss[gg] + e)
            new_qq.append(qq[gg] + e * e)
        return (*new_ss, *new_qq, ls + lbufs[b][f])

      res = lax.fori_loop(0, F, _accum, (zero,) * (2 * KG + 1))
      ss = res[:KG]
      qq = res[KG:2 * KG]
      ls = res[2 * KG]
      r = zero
      for gg in range(KG):
        r = r + (ss[gg] * ss[gg] - qq[gg])
      v = 0.5 * r + ls
      total = bias0
      for lane_i in range(_L):
        total = total + v[lane_i]
      lane = lax.rem(j, _L)
      resvec = jnp.where(lane_iota == lane, total, resvec)
      if (b + 1) % _L == 0 or _NBUF < _L and b == _NBUF - 1:

        @pl.when(lane == _L - 1)
        def _():
          out_v[pl.ds(j - (_L - 1), _L)] = resvec

      nj = j + _NBUF

      @pl.when(nj < BPW)
      def _():
        _fire(nj, b)

    return resvec

  lax.fori_loop(0, BPW // _NBUF, _gstep, zero)
  pltpu.sync_copy(out_v, out_hbm.at[pl.ds(base, BPW)])


def kernel(cat_features, emb_table, lin_table, bias):
  B, F = cat_features.shape
  V, K = emb_table.shape
  NW = _NC * _NS
  BPW = B // NW
  FP = -(-F // 8) * 8

  cat_pad = jnp.pad(cat_features, ((0, 0), (0, FP - F)))
  lin_pad = jnp.pad(lin_table, ((0, 0), (0, _L - lin_table.shape[1])))
  bias_pad = jnp.pad(bias, (0, _L - bias.shape[0]))

  mesh = plsc.VectorSubcoreMesh(core_axis_name="c", subcore_axis_name="s")
  scratch = [
      pltpu.VMEM((2 * _CHUNK, FP), jnp.int32),
      pltpu.VMEM((BPW,), jnp.float32),
      pltpu.VMEM((_L,), jnp.float32),
  ]
  scratch += [pltpu.VMEM((FP, K), jnp.bfloat16) for _ in range(_NBUF)]
  scratch += [pltpu.VMEM((FP, _L), jnp.float32) for _ in range(_NBUF)]
  scratch += [pltpu.SemaphoreType.DMA for _ in range(2 * _NBUF)]

  body = functools.partial(_fm_body, F, FP, K, BPW)
  out = pl.kernel(
      body,
      out_type=jax.ShapeDtypeStruct((B,), jnp.float32),
      mesh=mesh,
      scratch_types=scratch,
      compiler_params=pltpu.CompilerParams(
          use_tc_tiling_on_sc=False, needs_layout_passes=False),
  )(cat_pad, emb_table.astype(jnp.bfloat16), lin_pad, bias_pad)
  return out.reshape(B, 1)
